# baseline (speedup 1.0000x reference)
import jax
import jax.numpy as jnp
from jax import lax
from jax.experimental import pallas as pl
from jax.experimental.pallas import tpu as pltpu
from jax.experimental.pallas import tpu_sc as plsc

N_ROWS = 1048576
N_BLOCKS = N_ROWS // 128
NC, NS = 2, 16
NW = NC * NS
BLOCKS_PER_W = N_BLOCKS // NW
CHUNK_BLOCKS = 32
N_CHUNKS = BLOCKS_PER_W // CHUNK_BLOCKS
NBUF = 4
OUT_CHUNK_W = CHUNK_BLOCKS * 512
TOTAL_W = N_ROWS * 4


def _sc_body(x_hbm, w_hbm, out_hbm, in_bufs, out_bufs, w_buf, sem_in, sem_out):
    cid = lax.axis_index("c")
    sid = lax.axis_index("s")
    wid = sid * NC + cid

    blk0 = wid * BLOCKS_PER_W

    def start_in(k):
        b0 = blk0 + k * CHUNK_BLOCKS
        return pltpu.async_copy(
            x_hbm.at[pl.ds(b0, CHUNK_BLOCKS), pl.ds(0, 384)],
            in_bufs[k % NBUF],
            sem_in,
        )

    def start_out(k):
        off = (blk0 + k * CHUNK_BLOCKS) * 512
        return pltpu.async_copy(
            out_bufs[k % NBUF], out_hbm.at[pl.ds(off, OUT_CHUNK_W)], sem_out
        )

    in_dmas = [start_in(0), start_in(1)]
    out_dmas = []

    pltpu.sync_copy(w_hbm, w_buf)
    wvec = w_buf[...]
    w0 = jnp.take(wvec, jnp.full((16,), 0, jnp.int32))
    w1 = jnp.take(wvec, jnp.full((16,), 1, jnp.int32))
    w2 = jnp.take(wvec, jnp.full((16,), 2, jnp.int32))
    zero = jnp.zeros((16,), jnp.float32)
    for k in range(N_CHUNKS):
        if k + 2 < N_CHUNKS:
            in_dmas.append(start_in(k + 2))
        if k >= NBUF:
            out_dmas[k - NBUF].wait()
        in_dmas[k].wait()
        in_buf = in_bufs[k % NBUF]
        out_buf = out_bufs[k % NBUF]

        @plsc.parallel_loop(0, CHUNK_BLOCKS, unroll=2)
        def _(b):
            p = b * 512
            for j in range(8):
                q = j * 16
                out_buf[pl.ds(p + q, 16)] = zero
                out_buf[pl.ds(p + q + 128, 16)] = w1 * in_buf[b, pl.ds(q + 128, 16)]
                out_buf[pl.ds(p + q + 256, 16)] = w0 * in_buf[b, pl.ds(q, 16)]
                out_buf[pl.ds(p + q + 384, 16)] = w2 * in_buf[b, pl.ds(q + 256, 16)]

        out_dmas.append(start_out(k))
    for d in out_dmas[N_CHUNKS - NBUF:]:
        d.wait()


@jax.jit
def _sc_spmm(x2d, w16):
    mesh = plsc.VectorSubcoreMesh(core_axis_name="c", subcore_axis_name="s")
    f = pl.kernel(
        _sc_body,
        out_type=jax.ShapeDtypeStruct((TOTAL_W,), jnp.float32),
        mesh=mesh,
        scratch_types=[
            [pltpu.VMEM((CHUNK_BLOCKS, 384), jnp.float32) for _ in range(NBUF)],
            [pltpu.VMEM((OUT_CHUNK_W,), jnp.float32) for _ in range(NBUF)],
            pltpu.VMEM((16,), jnp.float32),
            pltpu.SemaphoreType.DMA,
            pltpu.SemaphoreType.DMA,
        ],
        compiler_params=pltpu.CompilerParams(
            needs_layout_passes=False,
            use_tc_tiling_on_sc=False,
            disable_bounds_checks=True,
            disable_semaphore_checks=True,
        ),
    )
    return f(x2d, w16)


def kernel(x, weights):
    xp = jnp.pad(x, ((0, 0), (0, 1)))
    x2d = xp.T.reshape(4, N_BLOCKS, 128).transpose(1, 0, 2).reshape(N_BLOCKS, 512)
    yf = _sc_spmm(x2d, jnp.pad(weights.astype(jnp.float32), (0, 13)))
    return yf.reshape(N_BLOCKS, 4, 128).transpose(1, 0, 2).reshape(4, N_ROWS).T


# speedup vs baseline: 1.0319x; 1.0319x over previous
import jax
import jax.numpy as jnp
from jax import lax
from jax.experimental import pallas as pl
from jax.experimental.pallas import tpu as pltpu
from jax.experimental.pallas import tpu_sc as plsc

N_ROWS = 1048576
N_BLOCKS = N_ROWS // 128
NC, NS = 2, 16
NW = NC * NS
BLOCKS_PER_W = N_BLOCKS // NW
CHUNK_BLOCKS = 64
N_CHUNKS = BLOCKS_PER_W // CHUNK_BLOCKS
NBUF = 2
OUT_CHUNK_W = CHUNK_BLOCKS * 512
TOTAL_W = N_ROWS * 4


def _sc_body(x_hbm, w_hbm, out_hbm, in_bufs, out_bufs, w_buf, sem_in, sem_out):
    cid = lax.axis_index("c")
    sid = lax.axis_index("s")
    wid = sid * NC + cid

    blk0 = wid * BLOCKS_PER_W

    def start_in(k):
        b0 = blk0 + k * CHUNK_BLOCKS
        return pltpu.async_copy(
            x_hbm.at[pl.ds(b0, CHUNK_BLOCKS), pl.ds(0, 384)],
            in_bufs[k % NBUF],
            sem_in,
        )

    def start_out(k):
        off = (blk0 + k * CHUNK_BLOCKS) * 512
        return pltpu.async_copy(
            out_bufs[k % NBUF], out_hbm.at[pl.ds(off, OUT_CHUNK_W)], sem_out
        )

    in_dmas = [start_in(0)]
    out_dmas = []

    pltpu.sync_copy(w_hbm, w_buf)
    wvec = w_buf[...]
    w0 = jnp.take(wvec, jnp.full((16,), 0, jnp.int32))
    w1 = jnp.take(wvec, jnp.full((16,), 1, jnp.int32))
    w2 = jnp.take(wvec, jnp.full((16,), 2, jnp.int32))
    zero = jnp.zeros((16,), jnp.float32)
    for k in range(N_CHUNKS):
        if k + 1 < N_CHUNKS:
            in_dmas.append(start_in(k + 1))
        if k >= NBUF:
            out_dmas[k - NBUF].wait()
        in_dmas[k].wait()
        in_buf = in_bufs[k % NBUF]
        out_buf = out_bufs[k % NBUF]

        @plsc.parallel_loop(0, CHUNK_BLOCKS, unroll=2)
        def _(b):
            p = b * 512
            for j in range(8):
                q = j * 16
                out_buf[pl.ds(p + q, 16)] = zero
                out_buf[pl.ds(p + q + 128, 16)] = w1 * in_buf[b, pl.ds(q + 128, 16)]
                out_buf[pl.ds(p + q + 256, 16)] = w0 * in_buf[b, pl.ds(q, 16)]
                out_buf[pl.ds(p + q + 384, 16)] = w2 * in_buf[b, pl.ds(q + 256, 16)]

        out_dmas.append(start_out(k))
    for d in out_dmas[N_CHUNKS - NBUF:]:
        d.wait()


@jax.jit
def _sc_spmm(x2d, w16):
    mesh = plsc.VectorSubcoreMesh(core_axis_name="c", subcore_axis_name="s")
    f = pl.kernel(
        _sc_body,
        out_type=jax.ShapeDtypeStruct((TOTAL_W,), jnp.float32),
        mesh=mesh,
        scratch_types=[
            [pltpu.VMEM((CHUNK_BLOCKS, 384), jnp.float32) for _ in range(NBUF)],
            [pltpu.VMEM((OUT_CHUNK_W,), jnp.float32) for _ in range(NBUF)],
            pltpu.VMEM((16,), jnp.float32),
            pltpu.SemaphoreType.DMA,
            pltpu.SemaphoreType.DMA,
        ],
        compiler_params=pltpu.CompilerParams(
            needs_layout_passes=False,
            use_tc_tiling_on_sc=False,
            disable_bounds_checks=True,
            disable_semaphore_checks=True,
        ),
    )
    return f(x2d, w16)


def kernel(x, weights):
    xp = jnp.pad(x, ((0, 0), (0, 1)))
    x2d = xp.T.reshape(4, N_BLOCKS, 128).transpose(1, 0, 2).reshape(N_BLOCKS, 512)
    yf = _sc_spmm(x2d, jnp.pad(weights.astype(jnp.float32), (0, 13)))
    return yf.reshape(N_BLOCKS, 4, 128).transpose(1, 0, 2).reshape(4, N_ROWS).T


# speedup vs baseline: 1.0335x; 1.0016x over previous
import jax
import jax.numpy as jnp
from jax import lax
from jax.experimental import pallas as pl
from jax.experimental.pallas import tpu as pltpu
from jax.experimental.pallas import tpu_sc as plsc

N_ROWS = 1048576
N_BLOCKS = N_ROWS // 128
NC, NS = 2, 16
NW = NC * NS
BLOCKS_PER_W = N_BLOCKS // NW
CHUNK_BLOCKS = 64
N_CHUNKS = BLOCKS_PER_W // CHUNK_BLOCKS
NBUF = 2
OUT_CHUNK_W = CHUNK_BLOCKS * 512
TOTAL_W = N_ROWS * 4


def _sc_body(x_hbm, w_hbm, out_hbm, in_bufs, out_bufs, w_buf, sem_in, sem_out):
    cid = lax.axis_index("c")
    sid = lax.axis_index("s")
    wid = sid * NC + cid

    blk0 = wid * BLOCKS_PER_W

    def start_in(k):
        b0 = blk0 + k * CHUNK_BLOCKS
        return pltpu.async_copy(
            x_hbm.at[pl.ds(b0, CHUNK_BLOCKS), pl.ds(0, 384)],
            in_bufs[k % NBUF],
            sem_in,
        )

    def start_out(k):
        off = (blk0 + k * CHUNK_BLOCKS) * 512
        return pltpu.async_copy(
            out_bufs[k % NBUF], out_hbm.at[pl.ds(off, OUT_CHUNK_W)], sem_out
        )

    in_dmas = [start_in(0)]
    out_dmas = []

    pltpu.sync_copy(w_hbm, w_buf)
    wvec = w_buf[...]
    w0 = jnp.take(wvec, jnp.full((16,), 0, jnp.int32))
    w1 = jnp.take(wvec, jnp.full((16,), 1, jnp.int32))
    w2 = jnp.take(wvec, jnp.full((16,), 2, jnp.int32))
    zero = jnp.zeros((16,), jnp.float32)
    for k in range(N_CHUNKS):
        if k + 1 < N_CHUNKS:
            in_dmas.append(start_in(k + 1))
        if k >= NBUF:
            out_dmas[k - NBUF].wait()
        in_dmas[k].wait()
        in_buf = in_bufs[k % NBUF]
        out_buf = out_bufs[k % NBUF]

        @plsc.parallel_loop(0, CHUNK_BLOCKS, unroll=4)
        def _(b):
            p = b * 512
            for j in range(8):
                q = j * 16
                out_buf[pl.ds(p + q, 16)] = zero
                out_buf[pl.ds(p + q + 128, 16)] = w1 * in_buf[b, pl.ds(q + 128, 16)]
                out_buf[pl.ds(p + q + 256, 16)] = w0 * in_buf[b, pl.ds(q, 16)]
                out_buf[pl.ds(p + q + 384, 16)] = w2 * in_buf[b, pl.ds(q + 256, 16)]

        out_dmas.append(start_out(k))
    for d in out_dmas[N_CHUNKS - NBUF:]:
        d.wait()


@jax.jit
def _sc_spmm(x2d, w16):
    mesh = plsc.VectorSubcoreMesh(core_axis_name="c", subcore_axis_name="s")
    f = pl.kernel(
        _sc_body,
        out_type=jax.ShapeDtypeStruct((TOTAL_W,), jnp.float32),
        mesh=mesh,
        scratch_types=[
            [pltpu.VMEM((CHUNK_BLOCKS, 384), jnp.float32) for _ in range(NBUF)],
            [pltpu.VMEM((OUT_CHUNK_W,), jnp.float32) for _ in range(NBUF)],
            pltpu.VMEM((16,), jnp.float32),
            pltpu.SemaphoreType.DMA,
            pltpu.SemaphoreType.DMA,
        ],
        compiler_params=pltpu.CompilerParams(
            needs_layout_passes=False,
            use_tc_tiling_on_sc=False,
            disable_bounds_checks=True,
            disable_semaphore_checks=True,
        ),
    )
    return f(x2d, w16)


def kernel(x, weights):
    xp = jnp.pad(x, ((0, 0), (0, 1)))
    x2d = xp.T.reshape(4, N_BLOCKS, 128).transpose(1, 0, 2).reshape(N_BLOCKS, 512)
    yf = _sc_spmm(x2d, jnp.pad(weights.astype(jnp.float32), (0, 13)))
    return yf.reshape(N_BLOCKS, 4, 128).transpose(1, 0, 2).reshape(4, N_ROWS).T
